# baseline (speedup 1.0000x reference)
import jax
import jax.numpy as jnp
from jax import lax
from jax.experimental import pallas as pl
from jax.experimental.pallas import tpu as pltpu
from jax.experimental.pallas import tpu_sc as plsc

_V = 1000000
_VT = 7813
_NJ = 13
_BATCH = 16384
_K = 16
_KP = _K // 2
_NC = 2
_NS = 16
_NW = _NC * _NS
_BPW = _BATCH // _NW
_CHUNK = 128
_NCHUNK = _BPW // _CHUNK
_NFCH = _BPW * _KP // _CHUNK
_GROUPS = _BPW // _K
import numpy as np

_MASK_HI = np.uint32(0xFFFF0000)


def _pack_tc_kernel(t_ref, out_ref):
    lo = jax.lax.bitcast_convert_type(t_ref[0], jnp.uint32) >> 16
    hi = jax.lax.bitcast_convert_type(t_ref[1], jnp.uint32) & _MASK_HI
    words = jax.lax.bitcast_convert_type(lo | hi, jnp.float32)
    for j in range(_NJ):
        out_ref[pl.ds(j * _KP, _KP), :] = words[:, j * _CHUNK:(j + 1) * _CHUNK]


def _pack_table(t):
    tv = t.T.reshape(2, _KP, _V)
    packed = pl.pallas_call(
        _pack_tc_kernel,
        grid=(_VT // _NJ,),
        in_specs=[pl.BlockSpec((2, _KP, _NJ * _CHUNK), lambda j: (0, 0, j))],
        out_specs=pl.BlockSpec((_NJ * _KP, _CHUNK), lambda j: (j, 0)),
        out_shape=jax.ShapeDtypeStruct((_VT * _KP, _CHUNK), jnp.float32),
    )(tv)
    return packed.reshape(_VT * _KP * _CHUNK)


def _mf_sc_kernel(uidx_hbm, vidx_hbm, w_hbm, h_hbm, out_hbm,
                  uidx_v, vidx_v, fidx_v, gidx_v, urows_v, vrows_v, out_v,
                  sem):
    wid = lax.axis_index("s") * _NC + lax.axis_index("c")
    base = wid * _BPW

    pltpu.sync_copy(uidx_hbm.at[pl.ds(wid * _NCHUNK, _NCHUNK)], uidx_v)
    pltpu.sync_copy(vidx_hbm.at[pl.ds(wid * _NCHUNK, _NCHUNK)], vidx_v)

    def idx_body(cp, carry):
        cbase = cp * _CHUNK
        for j in range(_NCHUNK):
            for s in range(_CHUNK // _K):
                ids = uidx_v[j, pl.ds(s * _K, _K)]
                gds = vidx_v[j, pl.ds(s * _K, _K)]
                fidx_v[cp * _NCHUNK + j, pl.ds(s * _K, _K)] = (
                    cbase + (ids >> 7) * 1024 + (ids & 127))
                gidx_v[cp * _NCHUNK + j, pl.ds(s * _K, _K)] = (
                    cbase + (gds >> 7) * 1024 + (gds & 127))
        return carry

    lax.fori_loop(0, _KP, idx_body, 0)

    copies = []
    for j in range(_NFCH):
        copies.append(pltpu.async_copy(
            w_hbm.at[fidx_v.at[j]], urows_v.at[pl.ds(j * _CHUNK, _CHUNK)],
            sem))
        copies.append(pltpu.async_copy(
            h_hbm.at[gidx_v.at[j]], vrows_v.at[pl.ds(j * _CHUNK, _CHUNK)],
            sem))
    for cp in copies:
        cp.wait()

    mask_hi = jnp.full((_K,), -65536, jnp.int32)

    def group_body(g, carry):
        acc = jnp.zeros((_K,), jnp.float32)
        for cp in range(_KP):
            off = cp * _BPW + g * _K
            uw = plsc.bitcast(urows_v[pl.ds(off, _K)], jnp.int32)
            vw = plsc.bitcast(vrows_v[pl.ds(off, _K)], jnp.int32)
            ulo = plsc.bitcast(uw << 16, jnp.float32)
            vlo = plsc.bitcast(vw << 16, jnp.float32)
            uhi = plsc.bitcast(uw & mask_hi, jnp.float32)
            vhi = plsc.bitcast(vw & mask_hi, jnp.float32)
            acc = acc + ulo * vlo + uhi * vhi
        out_v[pl.ds(g * _K, _K)] = 1.0 / (1.0 + jnp.exp(-acc))
        return carry

    lax.fori_loop(0, _GROUPS, group_body, 0)

    pltpu.sync_copy(out_v, out_hbm.at[pl.ds(base, _BPW)])


@jax.jit
def kernel(x, W, H):
    uidx = x[:, 0].reshape(_NW * _NCHUNK, _CHUNK)
    vidx = x[:, 1].reshape(_NW * _NCHUNK, _CHUNK)
    wf = _pack_table(W)
    hf = _pack_table(H)
    mesh = plsc.VectorSubcoreMesh(core_axis_name="c", subcore_axis_name="s")
    run = pl.kernel(
        _mf_sc_kernel,
        out_type=jax.ShapeDtypeStruct((_BATCH,), jnp.float32),
        mesh=mesh,
        scratch_types=[
            pltpu.VMEM((_NCHUNK, _CHUNK), jnp.int32),
            pltpu.VMEM((_NCHUNK, _CHUNK), jnp.int32),
            pltpu.VMEM((_NFCH, _CHUNK), jnp.int32),
            pltpu.VMEM((_NFCH, _CHUNK), jnp.int32),
            pltpu.VMEM((_BPW * _KP,), jnp.float32),
            pltpu.VMEM((_BPW * _KP,), jnp.float32),
            pltpu.VMEM((_BPW,), jnp.float32),
            pltpu.SemaphoreType.DMA,
        ],
        compiler_params=pltpu.CompilerParams(
            needs_layout_passes=False, use_tc_tiling_on_sc=False),
    )
    return run(uidx, vidx, wf, hf)


# speedup vs baseline: 6.9134x; 6.9134x over previous
import jax
import jax.numpy as jnp
from jax import lax
from jax.experimental import pallas as pl
from jax.experimental.pallas import tpu as pltpu
from jax.experimental.pallas import tpu_sc as plsc

_V = 1000000
_VT = 7813
_NJ = 601
_BATCH = 16384
_K = 16
_KP = _K // 2
_NC = 2
_NS = 16
_NW = _NC * _NS
_BPW = _BATCH // _NW
_CHUNK = 128
_NCHUNK = _BPW // _CHUNK
_NFCH = _BPW * _KP // _CHUNK
_GROUPS = _BPW // _K
import numpy as np

_MASK_HI = np.uint32(0xFFFF0000)


def _pack_tc_kernel(t_ref, out_ref):
    lo = jax.lax.bitcast_convert_type(t_ref[0], jnp.uint32) >> 16
    hi = jax.lax.bitcast_convert_type(t_ref[1], jnp.uint32) & _MASK_HI
    words = jax.lax.bitcast_convert_type(lo | hi, jnp.float32)
    for j in range(_NJ):
        out_ref[pl.ds(j * _KP, _KP), :] = words[:, j * _CHUNK:(j + 1) * _CHUNK]


def _pack_table(t):
    tv = t.T.reshape(2, _KP, _V)
    packed = pl.pallas_call(
        _pack_tc_kernel,
        grid=(_VT // _NJ,),
        in_specs=[pl.BlockSpec((2, _KP, _NJ * _CHUNK), lambda j: (0, 0, j))],
        out_specs=pl.BlockSpec((_NJ * _KP, _CHUNK), lambda j: (j, 0)),
        out_shape=jax.ShapeDtypeStruct((_VT * _KP, _CHUNK), jnp.float32),
    )(tv)
    return packed.reshape(_VT * _KP * _CHUNK)


def _mf_sc_kernel(uidx_hbm, vidx_hbm, w_hbm, h_hbm, out_hbm,
                  uidx_v, vidx_v, fidx_v, gidx_v, urows_v, vrows_v, out_v,
                  sem):
    wid = lax.axis_index("s") * _NC + lax.axis_index("c")
    base = wid * _BPW

    pltpu.sync_copy(uidx_hbm.at[pl.ds(wid * _NCHUNK, _NCHUNK)], uidx_v)
    pltpu.sync_copy(vidx_hbm.at[pl.ds(wid * _NCHUNK, _NCHUNK)], vidx_v)

    def idx_body(cp, carry):
        cbase = cp * _CHUNK
        for j in range(_NCHUNK):
            for s in range(_CHUNK // _K):
                ids = uidx_v[j, pl.ds(s * _K, _K)]
                gds = vidx_v[j, pl.ds(s * _K, _K)]
                fidx_v[cp * _NCHUNK + j, pl.ds(s * _K, _K)] = (
                    cbase + (ids >> 7) * 1024 + (ids & 127))
                gidx_v[cp * _NCHUNK + j, pl.ds(s * _K, _K)] = (
                    cbase + (gds >> 7) * 1024 + (gds & 127))
        return carry

    lax.fori_loop(0, _KP, idx_body, 0)

    copies = []
    for j in range(_NFCH):
        copies.append(pltpu.async_copy(
            w_hbm.at[fidx_v.at[j]], urows_v.at[pl.ds(j * _CHUNK, _CHUNK)],
            sem))
        copies.append(pltpu.async_copy(
            h_hbm.at[gidx_v.at[j]], vrows_v.at[pl.ds(j * _CHUNK, _CHUNK)],
            sem))
    for cp in copies:
        cp.wait()

    mask_hi = jnp.full((_K,), -65536, jnp.int32)

    def group_body(g, carry):
        acc = jnp.zeros((_K,), jnp.float32)
        for cp in range(_KP):
            off = cp * _BPW + g * _K
            uw = plsc.bitcast(urows_v[pl.ds(off, _K)], jnp.int32)
            vw = plsc.bitcast(vrows_v[pl.ds(off, _K)], jnp.int32)
            ulo = plsc.bitcast(uw << 16, jnp.float32)
            vlo = plsc.bitcast(vw << 16, jnp.float32)
            uhi = plsc.bitcast(uw & mask_hi, jnp.float32)
            vhi = plsc.bitcast(vw & mask_hi, jnp.float32)
            acc = acc + ulo * vlo + uhi * vhi
        out_v[pl.ds(g * _K, _K)] = 1.0 / (1.0 + jnp.exp(-acc))
        return carry

    lax.fori_loop(0, _GROUPS, group_body, 0)

    pltpu.sync_copy(out_v, out_hbm.at[pl.ds(base, _BPW)])


@jax.jit
def kernel(x, W, H):
    uidx = x[:, 0].reshape(_NW * _NCHUNK, _CHUNK)
    vidx = x[:, 1].reshape(_NW * _NCHUNK, _CHUNK)
    wf = _pack_table(W)
    hf = _pack_table(H)
    mesh = plsc.VectorSubcoreMesh(core_axis_name="c", subcore_axis_name="s")
    run = pl.kernel(
        _mf_sc_kernel,
        out_type=jax.ShapeDtypeStruct((_BATCH,), jnp.float32),
        mesh=mesh,
        scratch_types=[
            pltpu.VMEM((_NCHUNK, _CHUNK), jnp.int32),
            pltpu.VMEM((_NCHUNK, _CHUNK), jnp.int32),
            pltpu.VMEM((_NFCH, _CHUNK), jnp.int32),
            pltpu.VMEM((_NFCH, _CHUNK), jnp.int32),
            pltpu.VMEM((_BPW * _KP,), jnp.float32),
            pltpu.VMEM((_BPW * _KP,), jnp.float32),
            pltpu.VMEM((_BPW,), jnp.float32),
            pltpu.SemaphoreType.DMA,
        ],
        compiler_params=pltpu.CompilerParams(
            needs_layout_passes=False, use_tc_tiling_on_sc=False),
    )
    return run(uidx, vidx, wf, hf)


# speedup vs baseline: 7.1053x; 1.0278x over previous
import jax
import jax.numpy as jnp
from jax import lax
from jax.experimental import pallas as pl
from jax.experimental.pallas import tpu as pltpu
from jax.experimental.pallas import tpu_sc as plsc

_V = 1000000
_VT = 7813
_NJ = 601
_BATCH = 16384
_K = 16
_KP = _K // 2
_NC = 2
_NS = 16
_NW = _NC * _NS
_BPW = _BATCH // _NW
_CHUNK = 128
_NCHUNK = _BPW // _CHUNK
_NFCH = _BPW * _KP // _CHUNK
_GROUPS = _BPW // _K
import numpy as np

_MASK_HI = np.uint32(0xFFFF0000)


def _pack_tc_kernel(t_ref, out_ref):
    lo = jax.lax.bitcast_convert_type(t_ref[0], jnp.uint32) >> 16
    hi = jax.lax.bitcast_convert_type(t_ref[1], jnp.uint32) & _MASK_HI
    words = jax.lax.bitcast_convert_type(lo | hi, jnp.float32)
    for j in range(_NJ):
        out_ref[pl.ds(j * _KP, _KP), :] = words[:, j * _CHUNK:(j + 1) * _CHUNK]


def _pack_table(t):
    tv = t.T.reshape(2, _KP, _V)
    packed = pl.pallas_call(
        _pack_tc_kernel,
        grid=(_VT // _NJ,),
        in_specs=[pl.BlockSpec((2, _KP, _NJ * _CHUNK), lambda j: (0, 0, j))],
        out_specs=pl.BlockSpec((_NJ * _KP, _CHUNK), lambda j: (j, 0)),
        out_shape=jax.ShapeDtypeStruct((_VT * _KP, _CHUNK), jnp.float32),
    )(tv)
    return packed.reshape(_VT * _KP * _CHUNK)


def _build_idx(idx_v, fidx_v):
    def idx_body(cp, carry):
        cbase = cp * _CHUNK
        for j in range(_NCHUNK):
            for s in range(_CHUNK // _K):
                ids = idx_v[j, pl.ds(s * _K, _K)]
                fidx_v[cp * _NCHUNK + j, pl.ds(s * _K, _K)] = (
                    cbase + (ids >> 7) * 1024 + (ids & 127))
        return carry

    lax.fori_loop(0, _KP, idx_body, 0)


def _gather_words(table_hbm, fidx_v, rows_v, sem):
    copies = [
        pltpu.async_copy(
            table_hbm.at[fidx_v.at[j]],
            rows_v.at[pl.ds(j * _CHUNK, _CHUNK)], sem)
        for j in range(_NFCH)
    ]
    for cp in copies:
        cp.wait()


def _u_sc_kernel(uidx_hbm, w_hbm, uw_hbm, uidx_v, fidx_v, urows_v, sem):
    wid = lax.axis_index("s") * _NC + lax.axis_index("c")
    pltpu.sync_copy(uidx_hbm.at[pl.ds(wid * _NCHUNK, _NCHUNK)], uidx_v)
    _build_idx(uidx_v, fidx_v)
    _gather_words(w_hbm, fidx_v, urows_v, sem)
    pltpu.sync_copy(urows_v, uw_hbm.at[pl.ds(wid * _BPW * _KP, _BPW * _KP)])


def _v_sc_kernel(vidx_hbm, h_hbm, uw_hbm, out_hbm,
                 vidx_v, gidx_v, urows_v, vrows_v, out_v, sem):
    wid = lax.axis_index("s") * _NC + lax.axis_index("c")
    base = wid * _BPW

    pltpu.sync_copy(vidx_hbm.at[pl.ds(wid * _NCHUNK, _NCHUNK)], vidx_v)
    _build_idx(vidx_v, gidx_v)
    pltpu.sync_copy(uw_hbm.at[pl.ds(wid * _BPW * _KP, _BPW * _KP)], urows_v)
    _gather_words(h_hbm, gidx_v, vrows_v, sem)

    mask_hi = jnp.full((_K,), -65536, jnp.int32)

    def group_body(g, carry):
        acc = jnp.zeros((_K,), jnp.float32)
        for cp in range(_KP):
            off = cp * _BPW + g * _K
            uw = plsc.bitcast(urows_v[pl.ds(off, _K)], jnp.int32)
            vw = plsc.bitcast(vrows_v[pl.ds(off, _K)], jnp.int32)
            ulo = plsc.bitcast(uw << 16, jnp.float32)
            vlo = plsc.bitcast(vw << 16, jnp.float32)
            uhi = plsc.bitcast(uw & mask_hi, jnp.float32)
            vhi = plsc.bitcast(vw & mask_hi, jnp.float32)
            acc = acc + ulo * vlo + uhi * vhi
        out_v[pl.ds(g * _K, _K)] = 1.0 / (1.0 + jnp.exp(-acc))
        return carry

    lax.fori_loop(0, _GROUPS, group_body, 0)

    pltpu.sync_copy(out_v, out_hbm.at[pl.ds(base, _BPW)])


@jax.jit
def kernel(x, W, H):
    uidx = x[:, 0].reshape(_NW * _NCHUNK, _CHUNK)
    vidx = x[:, 1].reshape(_NW * _NCHUNK, _CHUNK)
    mesh = plsc.VectorSubcoreMesh(core_axis_name="c", subcore_axis_name="s")
    params = pltpu.CompilerParams(
        needs_layout_passes=False, use_tc_tiling_on_sc=False)

    wf = _pack_table(W)
    run_u = pl.kernel(
        _u_sc_kernel,
        out_type=jax.ShapeDtypeStruct((_BATCH * _KP,), jnp.float32),
        mesh=mesh,
        scratch_types=[
            pltpu.VMEM((_NCHUNK, _CHUNK), jnp.int32),
            pltpu.VMEM((_NFCH, _CHUNK), jnp.int32),
            pltpu.VMEM((_BPW * _KP,), jnp.float32),
            pltpu.SemaphoreType.DMA,
        ],
        compiler_params=params,
    )
    uw = run_u(uidx, wf)

    hf = _pack_table(H)
    run_v = pl.kernel(
        _v_sc_kernel,
        out_type=jax.ShapeDtypeStruct((_BATCH,), jnp.float32),
        mesh=mesh,
        scratch_types=[
            pltpu.VMEM((_NCHUNK, _CHUNK), jnp.int32),
            pltpu.VMEM((_NFCH, _CHUNK), jnp.int32),
            pltpu.VMEM((_BPW * _KP,), jnp.float32),
            pltpu.VMEM((_BPW * _KP,), jnp.float32),
            pltpu.VMEM((_BPW,), jnp.float32),
            pltpu.SemaphoreType.DMA,
        ],
        compiler_params=params,
    )
    return run_v(vidx, hf, uw)


# speedup vs baseline: 7.1639x; 1.0082x over previous
import jax
import jax.numpy as jnp
from jax import lax
from jax.experimental import pallas as pl
from jax.experimental.pallas import tpu as pltpu
from jax.experimental.pallas import tpu_sc as plsc

_V = 1000000
_VT = 7813
_NJ = 977
_BATCH = 16384
_K = 16
_KP = _K // 2
_NC = 2
_NS = 16
_NW = _NC * _NS
_BPW = _BATCH // _NW
_CHUNK = 128
_NCHUNK = _BPW // _CHUNK
_NFCH = _BPW * _KP // _CHUNK
_GROUPS = _BPW // _K
import numpy as np

_MASK_HI = np.uint32(0xFFFF0000)


def _pack_tc_kernel(t_ref, out_ref):
    lo = jax.lax.bitcast_convert_type(t_ref[0], jnp.uint32) >> 16
    hi = jax.lax.bitcast_convert_type(t_ref[1], jnp.uint32) & _MASK_HI
    words = jax.lax.bitcast_convert_type(lo | hi, jnp.float32)
    for j in range(_NJ):
        out_ref[pl.ds(j * _KP, _KP), :] = words[:, j * _CHUNK:(j + 1) * _CHUNK]


def _pack_table(t):
    tv = t.T.reshape(2, _KP, _V)
    packed = pl.pallas_call(
        _pack_tc_kernel,
        grid=(-(-_VT // _NJ),),
        in_specs=[pl.BlockSpec((2, _KP, _NJ * _CHUNK), lambda j: (0, 0, j))],
        out_specs=pl.BlockSpec((_NJ * _KP, _CHUNK), lambda j: (j, 0)),
        out_shape=jax.ShapeDtypeStruct((_VT * _KP, _CHUNK), jnp.float32),
    )(tv)
    return packed.reshape(_VT * _KP * _CHUNK)


def _build_idx(idx_v, fidx_v):
    def idx_body(cp, carry):
        cbase = cp * _CHUNK
        for j in range(_NCHUNK):
            for s in range(_CHUNK // _K):
                ids = idx_v[j, pl.ds(s * _K, _K)]
                fidx_v[cp * _NCHUNK + j, pl.ds(s * _K, _K)] = (
                    cbase + (ids >> 7) * 1024 + (ids & 127))
        return carry

    lax.fori_loop(0, _KP, idx_body, 0)


def _gather_words(table_hbm, fidx_v, rows_v, sem):
    copies = [
        pltpu.async_copy(
            table_hbm.at[fidx_v.at[j]],
            rows_v.at[pl.ds(j * _CHUNK, _CHUNK)], sem)
        for j in range(_NFCH)
    ]
    for cp in copies:
        cp.wait()


def _u_sc_kernel(uidx_hbm, w_hbm, uw_hbm, uidx_v, fidx_v, urows_v, sem):
    wid = lax.axis_index("s") * _NC + lax.axis_index("c")
    pltpu.sync_copy(uidx_hbm.at[pl.ds(wid * _NCHUNK, _NCHUNK)], uidx_v)
    _build_idx(uidx_v, fidx_v)
    _gather_words(w_hbm, fidx_v, urows_v, sem)
    pltpu.sync_copy(urows_v, uw_hbm.at[pl.ds(wid * _BPW * _KP, _BPW * _KP)])


def _v_sc_kernel(vidx_hbm, h_hbm, uw_hbm, out_hbm,
                 vidx_v, gidx_v, urows_v, vrows_v, out_v, sem):
    wid = lax.axis_index("s") * _NC + lax.axis_index("c")
    base = wid * _BPW

    pltpu.sync_copy(vidx_hbm.at[pl.ds(wid * _NCHUNK, _NCHUNK)], vidx_v)
    _build_idx(vidx_v, gidx_v)
    pltpu.sync_copy(uw_hbm.at[pl.ds(wid * _BPW * _KP, _BPW * _KP)], urows_v)
    _gather_words(h_hbm, gidx_v, vrows_v, sem)

    mask_hi = jnp.full((_K,), -65536, jnp.int32)

    def group_body(g, carry):
        acc = jnp.zeros((_K,), jnp.float32)
        for cp in range(_KP):
            off = cp * _BPW + g * _K
            uw = plsc.bitcast(urows_v[pl.ds(off, _K)], jnp.int32)
            vw = plsc.bitcast(vrows_v[pl.ds(off, _K)], jnp.int32)
            ulo = plsc.bitcast(uw << 16, jnp.float32)
            vlo = plsc.bitcast(vw << 16, jnp.float32)
            uhi = plsc.bitcast(uw & mask_hi, jnp.float32)
            vhi = plsc.bitcast(vw & mask_hi, jnp.float32)
            acc = acc + ulo * vlo + uhi * vhi
        out_v[pl.ds(g * _K, _K)] = 1.0 / (1.0 + jnp.exp(-acc))
        return carry

    lax.fori_loop(0, _GROUPS, group_body, 0)

    pltpu.sync_copy(out_v, out_hbm.at[pl.ds(base, _BPW)])


@jax.jit
def kernel(x, W, H):
    uidx = x[:, 0].reshape(_NW * _NCHUNK, _CHUNK)
    vidx = x[:, 1].reshape(_NW * _NCHUNK, _CHUNK)
    mesh = plsc.VectorSubcoreMesh(core_axis_name="c", subcore_axis_name="s")
    params = pltpu.CompilerParams(
        needs_layout_passes=False, use_tc_tiling_on_sc=False)

    wf = _pack_table(W)
    run_u = pl.kernel(
        _u_sc_kernel,
        out_type=jax.ShapeDtypeStruct((_BATCH * _KP,), jnp.float32),
        mesh=mesh,
        scratch_types=[
            pltpu.VMEM((_NCHUNK, _CHUNK), jnp.int32),
            pltpu.VMEM((_NFCH, _CHUNK), jnp.int32),
            pltpu.VMEM((_BPW * _KP,), jnp.float32),
            pltpu.SemaphoreType.DMA,
        ],
        compiler_params=params,
    )
    uw = run_u(uidx, wf)

    hf = _pack_table(H)
    run_v = pl.kernel(
        _v_sc_kernel,
        out_type=jax.ShapeDtypeStruct((_BATCH,), jnp.float32),
        mesh=mesh,
        scratch_types=[
            pltpu.VMEM((_NCHUNK, _CHUNK), jnp.int32),
            pltpu.VMEM((_NFCH, _CHUNK), jnp.int32),
            pltpu.VMEM((_BPW * _KP,), jnp.float32),
            pltpu.VMEM((_BPW * _KP,), jnp.float32),
            pltpu.VMEM((_BPW,), jnp.float32),
            pltpu.SemaphoreType.DMA,
        ],
        compiler_params=params,
    )
    return run_v(vidx, hf, uw)


# speedup vs baseline: 7.1699x; 1.0008x over previous
import jax
import jax.numpy as jnp
import numpy as np
from jax import lax
from jax.experimental import pallas as pl
from jax.experimental.pallas import tpu as pltpu
from jax.experimental.pallas import tpu_sc as plsc

_V = 1000000
_VT = 7813
_NJ = 977
_BATCH = 16384
_K = 16
_KP = _K // 2
_NC = 2
_NS = 16
_NW = _NC * _NS
_BPW = _BATCH // _NW
_CHUNK = 128
_NCHUNK = _BPW // _CHUNK
_NFCH = _BPW * _KP // _CHUNK
_GROUPS = _BPW // _K
_MASK_HI = np.uint32(0xFFFF0000)


def _pack_tc_kernel(t_ref, out_ref):
    lo = jax.lax.bitcast_convert_type(t_ref[0], jnp.uint32) >> 16
    hi = jax.lax.bitcast_convert_type(t_ref[1], jnp.uint32) & _MASK_HI
    words = jax.lax.bitcast_convert_type(lo | hi, jnp.float32)
    for j in range(_NJ):
        out_ref[pl.ds(j * _KP, _KP), :] = words[:, j * _CHUNK:(j + 1) * _CHUNK]


def _pack_table(t):
    tv = t.T.reshape(2, _KP, _V)
    packed = pl.pallas_call(
        _pack_tc_kernel,
        grid=(-(-_VT // _NJ),),
        in_specs=[pl.BlockSpec((2, _KP, _NJ * _CHUNK), lambda j: (0, 0, j))],
        out_specs=pl.BlockSpec((_NJ * _KP, _CHUNK), lambda j: (j, 0)),
        out_shape=jax.ShapeDtypeStruct((_VT * _KP, _CHUNK), jnp.float32),
    )(tv)
    return packed.reshape(_VT * _KP * _CHUNK)


def _build_idx(idx_v, fidx_v):
    def idx_body(cp, carry):
        cbase = cp * _CHUNK
        for j in range(_NCHUNK):
            for s in range(_CHUNK // _K):
                ids = idx_v[j, pl.ds(s * _K, _K)]
                fidx_v[cp * _NCHUNK + j, pl.ds(s * _K, _K)] = (
                    cbase + (ids >> 7) * 1024 + (ids & 127))
        return carry

    lax.fori_loop(0, _KP, idx_body, 0)


def _gather_words(table_hbm, fidx_v, rows_v, sem):
    copies = [
        pltpu.async_copy(
            table_hbm.at[fidx_v.at[j]],
            rows_v.at[pl.ds(j * _CHUNK, _CHUNK)], sem)
        for j in range(_NFCH)
    ]
    for cp in copies:
        cp.wait()


def _u_sc_kernel(uidx_hbm, w_hbm, uw_hbm, uidx_v, fidx_v, urows_v, sem):
    wid = lax.axis_index("s") * _NC + lax.axis_index("c")
    pltpu.sync_copy(uidx_hbm.at[pl.ds(wid * _NCHUNK, _NCHUNK)], uidx_v)
    _build_idx(uidx_v, fidx_v)
    _gather_words(w_hbm, fidx_v, urows_v, sem)
    pltpu.sync_copy(urows_v, uw_hbm.at[pl.ds(wid * _BPW * _KP, _BPW * _KP)])


def _v_sc_kernel(vidx_hbm, h_hbm, uw_hbm, out_hbm,
                 vidx_v, gidx_v, urows_v, vrows_v, out_v, sem):
    wid = lax.axis_index("s") * _NC + lax.axis_index("c")
    base = wid * _BPW

    pltpu.sync_copy(vidx_hbm.at[pl.ds(wid * _NCHUNK, _NCHUNK)], vidx_v)
    _build_idx(vidx_v, gidx_v)
    pltpu.sync_copy(uw_hbm.at[pl.ds(wid * _BPW * _KP, _BPW * _KP)], urows_v)
    _gather_words(h_hbm, gidx_v, vrows_v, sem)

    mask_hi = jnp.full((_K,), -65536, jnp.int32)

    def group_body(g, carry):
        acc = jnp.zeros((_K,), jnp.float32)
        for cp in range(_KP):
            off = cp * _BPW + g * _K
            uw = plsc.bitcast(urows_v[pl.ds(off, _K)], jnp.int32)
            vw = plsc.bitcast(vrows_v[pl.ds(off, _K)], jnp.int32)
            ulo = plsc.bitcast(uw << 16, jnp.float32)
            vlo = plsc.bitcast(vw << 16, jnp.float32)
            uhi = plsc.bitcast(uw & mask_hi, jnp.float32)
            vhi = plsc.bitcast(vw & mask_hi, jnp.float32)
            acc = acc + ulo * vlo + uhi * vhi
        out_v[pl.ds(g * _K, _K)] = 1.0 / (1.0 + jnp.exp(-acc))
        return carry

    lax.fori_loop(0, _GROUPS, group_body, 0)

    pltpu.sync_copy(out_v, out_hbm.at[pl.ds(base, _BPW)])


@jax.jit
def kernel(x, W, H):
    uidx = x[:, 0].reshape(_NW * _NCHUNK, _CHUNK)
    vidx = x[:, 1].reshape(_NW * _NCHUNK, _CHUNK)
    mesh = plsc.VectorSubcoreMesh(core_axis_name="c", subcore_axis_name="s")
    params = pltpu.CompilerParams(
        needs_layout_passes=False, use_tc_tiling_on_sc=False)

    wf = _pack_table(W)
    run_u = pl.kernel(
        _u_sc_kernel,
        out_type=jax.ShapeDtypeStruct((_BATCH * _KP,), jnp.float32),
        mesh=mesh,
        scratch_types=[
            pltpu.VMEM((_NCHUNK, _CHUNK), jnp.int32),
            pltpu.VMEM((_NFCH, _CHUNK), jnp.int32),
            pltpu.VMEM((_BPW * _KP,), jnp.float32),
            pltpu.SemaphoreType.DMA,
        ],
        compiler_params=params,
    )
    uw = run_u(uidx, wf)

    hf = _pack_table(H)
    run_v = pl.kernel(
        _v_sc_kernel,
        out_type=jax.ShapeDtypeStruct((_BATCH,), jnp.float32),
        mesh=mesh,
        scratch_types=[
            pltpu.VMEM((_NCHUNK, _CHUNK), jnp.int32),
            pltpu.VMEM((_NFCH, _CHUNK), jnp.int32),
            pltpu.VMEM((_BPW * _KP,), jnp.float32),
            pltpu.VMEM((_BPW * _KP,), jnp.float32),
            pltpu.VMEM((_BPW,), jnp.float32),
            pltpu.SemaphoreType.DMA,
        ],
        compiler_params=params,
    )
    return run_v(vidx, hf, uw)
